# Initial kernel scaffold; baseline (speedup 1.0000x reference)
import jax
import jax.numpy as jnp
from jax.experimental import pallas as pl


def kernel(h, edge_index, e, snorm_n, snorm_e, W_embed, b_embed, A_coef, Wf, bf, gamma, beta, W1, b1, W2, b2, W3, b3):
    raise NotImplementedError("write your pallas kernel here")


# speedup vs baseline: 3.4455x; 3.4455x over previous
import functools

import jax
import jax.numpy as jnp
from jax import lax
from jax.experimental import pallas as pl
from jax.experimental.pallas import tpu as pltpu
from jax.experimental.pallas import tpu_sc as plsc

N = 10000
E = 320000
H = 128
NTILES = 16
C = 128
CHUNKS = 157
E_PAD = NTILES * C * CHUNKS
ROWS_PAD = 10112
ZROWS = 632
ECHUNK = 80
ECHUNKS = N // ECHUNK
EROWS = 625
DEG_PAD = 10240
F32 = jnp.float32


def _sc_mesh():
    return plsc.VectorSubcoreMesh(
        core_axis_name="c", subcore_axis_name="s", num_cores=1)


@functools.partial(
    pl.kernel,
    out_type=jax.ShapeDtypeStruct((DEG_PAD,), F32),
    mesh=_sc_mesh(),
    scratch_types=dict(
        acc=pltpu.VMEM_SHARED((DEG_PAD,), F32),
        didx=pltpu.VMEM((C,), jnp.int32),
        ones=pltpu.VMEM((C,), F32),
        dbuf=pltpu.VMEM((640,), F32),
    ),
)
def _deg_kernel(dstdeg_hbm, zeros1_hbm, deg_out, acc, didx, ones, dbuf):
    t = lax.axis_index("s")
    pltpu.sync_copy(zeros1_hbm.at[pl.ds(t * 640, 640)],
                    acc.at[pl.ds(t * 640, 640)])
    for j in range(C // 16):
        ones[pl.ds(j * 16, 16)] = jnp.ones((16,), F32)
    plsc.subcore_barrier()

    def body(i, carry):
        base = pl.multiple_of((t * CHUNKS + i) * C, C)
        pltpu.sync_copy(dstdeg_hbm.at[pl.ds(base, C)], didx)
        pltpu.sync_copy(ones, acc.at[didx], add=True)
        return carry

    lax.fori_loop(0, CHUNKS, body, 0)
    plsc.subcore_barrier()
    pltpu.sync_copy(acc.at[pl.ds(t * 640, 640)], dbuf)
    pltpu.sync_copy(dbuf, deg_out.at[pl.ds(t * 640, 640)])


@functools.partial(
    pl.kernel,
    out_type=(jax.ShapeDtypeStruct((N, H), F32),
              jax.ShapeDtypeStruct((N, H), F32)),
    mesh=_sc_mesh(),
    scratch_types=dict(
        acc=pltpu.VMEM_SHARED((ROWS_PAD, H), F32),
        sidx=pltpu.VMEM((C,), jnp.int32),
        didx=pltpu.VMEM((C,), jnp.int32),
        rows=pltpu.VMEM((C, H), F32),
        ebuf=pltpu.VMEM((ECHUNK, H), F32),
        rs2buf=pltpu.VMEM((ECHUNK, 16), F32),
        gsem=pltpu.SemaphoreType.DMA,
    ),
)
def _hop_kernel(w_hbm, src_hbm, dst_hbm, rs2X_hbm, zeros2_hbm,
                a_out, w_out, acc, sidx, didx, rows, ebuf, rs2buf,
                gsem):
    t = lax.axis_index("s")
    pltpu.sync_copy(zeros2_hbm.at[pl.ds(0, ZROWS)],
                    acc.at[pl.ds(pl.multiple_of(t * ZROWS, 8), ZROWS)])
    plsc.subcore_barrier()

    def body(i, carry):
        base = pl.multiple_of((t * CHUNKS + i) * C, C)
        pltpu.sync_copy(src_hbm.at[pl.ds(base, C)], sidx)
        pltpu.sync_copy(dst_hbm.at[pl.ds(base, C)], didx)
        pltpu.async_copy(w_hbm.at[sidx], rows, gsem).wait()
        pltpu.sync_copy(rows, acc.at[didx], add=True)
        return carry

    lax.fori_loop(0, CHUNKS, body, 0)
    plsc.subcore_barrier()

    for j in range((ECHUNKS + NTILES - 1) // NTILES):
        c = t + NTILES * j

        @pl.when(c < ECHUNKS)
        def _():
            row0 = pl.multiple_of(c * ECHUNK, 8)
            pltpu.sync_copy(acc.at[pl.ds(row0, ECHUNK)], ebuf)
            pltpu.sync_copy(rs2X_hbm.at[pl.ds(row0, ECHUNK)], rs2buf)
            pltpu.sync_copy(ebuf, a_out.at[pl.ds(row0, ECHUNK)])

            def rbody(r, carry):
                s2 = rs2buf[r, pl.ds(0, 16)]
                for jj in range(H // 16):
                    v = ebuf[r, pl.ds(jj * 16, 16)]
                    ebuf[r, pl.ds(jj * 16, 16)] = v * s2
                return carry

            lax.fori_loop(0, ECHUNK, rbody, 0)
            pltpu.sync_copy(ebuf, w_out.at[pl.ds(row0, ECHUNK)])


def _embed_body(h_ref, we_ref, be_ref, rs_ref, hcur_ref, w_ref):
    hcur = jnp.dot(h_ref[...], we_ref[...],
                   preferred_element_type=F32) + be_ref[...]
    hcur_ref[...] = hcur
    w_ref[...] = hcur * rs_ref[...]


def _embed(h, We, be, rs_col):
    return pl.pallas_call(
        _embed_body,
        out_shape=(jax.ShapeDtypeStruct((N, H), F32),
                   jax.ShapeDtypeStruct((N, H), F32)),
    )(h, We, be, rs_col)


def _layer_body(hin_ref, a1_ref, a2_ref, a3_ref, rs_ref, sn_ref, A_ref,
                Wf_ref, bf_ref, g_ref, b_ref, hout_ref, wout_ref):
    hin = hin_ref[...]
    rs = rs_ref[...]
    hs = (hin, a1_ref[...] * rs, a2_ref[...] * rs, a3_ref[...] * rs)
    hc = jnp.zeros((N, H), F32)
    for f in range(3):
        zf = (A_ref[f, 0] * hs[0] + A_ref[f, 1] * hs[1]
              + A_ref[f, 2] * hs[2] + A_ref[f, 3] * hs[3])
        hc = hc + jnp.dot(zf, Wf_ref[f], preferred_element_type=F32) \
            + bf_ref[f]
    hc = hc * (1.0 / 3.0) * sn_ref[...]
    mu = jnp.mean(hc, axis=0, keepdims=True)
    var = jnp.mean((hc - mu) ** 2, axis=0, keepdims=True)
    hc = (hc - mu) * lax.rsqrt(var + 1e-5) * g_ref[...] + b_ref[...]
    hc = jnp.maximum(hc, 0.0)
    hout = hc + hin
    hout_ref[...] = hout
    wout_ref[...] = hout * rs


def _layer(hin, a1, a2, a3, rs_col, snorm_n, A_l, Wf_l, bf_l, g_l, b_l):
    return pl.pallas_call(
        _layer_body,
        out_shape=(jax.ShapeDtypeStruct((N, H), F32),
                   jax.ShapeDtypeStruct((N, H), F32)),
        in_specs=[pl.BlockSpec(memory_space=pltpu.MemorySpace.VMEM)] * 6
        + [pl.BlockSpec(memory_space=pltpu.MemorySpace.SMEM)]
        + [pl.BlockSpec(memory_space=pltpu.MemorySpace.VMEM)] * 4,
        compiler_params=pltpu.CompilerParams(
            vmem_limit_bytes=100 * 1024 * 1024),
    )(hin, a1, a2, a3, rs_col, snorm_n, A_l, Wf_l, bf_l, g_l, b_l)


def _readout_body(hc_ref, w1_ref, b1_ref, w2_ref, b2_ref, w3_ref, b3_ref,
                  out_ref):
    hg = jnp.mean(hc_ref[...], axis=0, keepdims=True)
    x = jnp.maximum(jnp.dot(hg, w1_ref[...],
                            preferred_element_type=F32) + b1_ref[...], 0.0)
    x = jnp.maximum(jnp.dot(x, w2_ref[...],
                            preferred_element_type=F32) + b2_ref[...], 0.0)
    out_ref[...] = jnp.dot(x, w3_ref[...],
                           preferred_element_type=F32) + b3_ref[...]


def _readout(hc, W1, b1, W2, b2, W3, b3):
    return pl.pallas_call(
        _readout_body,
        out_shape=jax.ShapeDtypeStruct((1, 10), F32),
    )(hc, W1, b1, W2, b2, W3, b3)


def kernel(h, edge_index, e, snorm_n, snorm_e, W_embed, b_embed, A_coef,
           Wf, bf, gamma, beta, W1, b1, W2, b2, W3, b3):
    src = edge_index[0]
    dst = edge_index[1]
    pad = E_PAD - E
    i32 = jnp.int32
    srcF = jnp.concatenate([src, jnp.zeros((pad,), i32)])
    dstF = jnp.concatenate([dst, jnp.full((pad,), N, i32)])
    dd = dst + 15 * (dst // EROWS)
    dstDegF = jnp.concatenate([dd, jnp.full((pad,), DEG_PAD - 1, i32)])
    zeros1 = jnp.zeros((DEG_PAD,), F32)
    zeros2 = jnp.zeros((ZROWS, H), F32)

    deg1d = _deg_kernel(dstDegF, zeros1)
    deg = deg1d.reshape(NTILES, 640)[:, :EROWS].reshape(N)
    degc = jnp.maximum(deg, 1.0)
    rs = lax.rsqrt(degc)
    rs2 = 1.0 / degc
    rs_col = rs[:, None]
    rs2X = jnp.broadcast_to(rs2[:, None], (N, 16))

    hcur, w = _embed(h, W_embed, b_embed.reshape(1, H), rs_col)
    for l in range(4):
        hin = hcur
        a_list = []
        for _ in range(3):
            a, w = _hop_kernel(w, srcF, dstF, rs2X, zeros2)
            a_list.append(a)
        hcur, w = _layer(hin, a_list[0], a_list[1], a_list[2], rs_col,
                         snorm_n, A_coef[l], Wf[l], bf[l].reshape(3, 1, H),
                         gamma[l].reshape(1, H), beta[l].reshape(1, H))
    out = _readout(hcur, W1, b1.reshape(1, -1), W2, b2.reshape(1, -1),
                   W3, b3.reshape(1, -1))
    return out


# speedup vs baseline: 3.6864x; 1.0699x over previous
import functools

import jax
import jax.numpy as jnp
from jax import lax
from jax.experimental import pallas as pl
from jax.experimental.pallas import tpu as pltpu
from jax.experimental.pallas import tpu_sc as plsc

N = 10000
E = 320000
H = 128
NTILES = 16
NW = 32
C = 128
CHUNKS = 80
E_PAD = NW * C * CHUNKS
CROWS = NW * CHUNKS
BLK = 16
NBLK = CHUNKS // BLK
ROWS_PAD = 10112
ZROWS = 632
DEG_PAD = 10240
F32 = jnp.float32


@functools.partial(
    pl.kernel,
    out_type=(jax.ShapeDtypeStruct((N,), F32),
              jax.ShapeDtypeStruct((N,), F32)),
    mesh=plsc.VectorSubcoreMesh(
        core_axis_name="c", subcore_axis_name="s", num_cores=2),
    scratch_types=dict(
        acc=pltpu.VMEM_SHARED((DEG_PAD,), F32),
        dbigA=pltpu.VMEM((BLK, C), jnp.int32),
        dbigB=pltpu.VMEM((BLK, C), jnp.int32),
        ones=pltpu.VMEM((C,), F32),
        dbuf=pltpu.VMEM((ZROWS,), F32),
        isemA=pltpu.SemaphoreType.DMA,
        isemB=pltpu.SemaphoreType.DMA,
        ssem=pltpu.SemaphoreType.DMA,
    ),
)
def _deg_kernel(dst2_hbm, zeros1_hbm, deg0_out, deg1_out,
                acc, dbigA, dbigB, ones, dbuf, isemA, isemB, ssem):
    cid = lax.axis_index("c")
    t = lax.axis_index("s")
    wid = cid * NTILES + t
    pltpu.sync_copy(zeros1_hbm.at[pl.ds(t * 640, 640)],
                    acc.at[pl.ds(t * 640, 640)])
    for j in range(C // 16):
        ones[pl.ds(j * 16, 16)] = jnp.ones((16,), F32)
    plsc.subcore_barrier()

    bufs = (dbigA, dbigB)
    sems = (isemA, isemB)

    def ifetch(b, dbig, isem):
        row0 = pl.multiple_of(wid * CHUNKS + b * BLK, 8)
        pltpu.async_copy(dst2_hbm.at[pl.ds(row0, BLK)], dbig, isem)

    ifetch(0, bufs[0], sems[0])
    for b in range(NBLK):
        dbig, isem = bufs[b % 2], sems[b % 2]
        pltpu.make_async_copy(
            dst2_hbm.at[pl.ds(0, BLK)], dbig, isem).wait()
        if b + 1 < NBLK:
            ifetch(b + 1, bufs[(b + 1) % 2], sems[(b + 1) % 2])
        for k in range(BLK):
            pltpu.async_copy(ones, acc.at[dbig.at[k]], ssem, add=True)
        for k in range(BLK):
            pltpu.make_async_copy(ones, acc.at[dbig.at[0]], ssem).wait()

    plsc.subcore_barrier()
    off = pl.multiple_of(t * ZROWS, 8)

    @pl.when(t < NTILES - 1)
    def _():
        pltpu.sync_copy(acc.at[pl.ds(off, ZROWS)], dbuf)

        @pl.when(cid == 0)
        def _():
            pltpu.sync_copy(dbuf, deg0_out.at[pl.ds(off, ZROWS)])

        @pl.when(cid == 1)
        def _():
            pltpu.sync_copy(dbuf, deg1_out.at[pl.ds(off, ZROWS)])

    @pl.when(t == NTILES - 1)
    def _():
        off15 = pl.multiple_of((NTILES - 1) * ZROWS, 8)
        pltpu.sync_copy(acc.at[pl.ds(off15, 520)], dbuf.at[pl.ds(0, 520)])

        @pl.when(cid == 0)
        def _():
            pltpu.sync_copy(dbuf.at[pl.ds(0, 520)],
                            deg0_out.at[pl.ds(off15, 520)])

        @pl.when(cid == 1)
        def _():
            pltpu.sync_copy(dbuf.at[pl.ds(0, 520)],
                            deg1_out.at[pl.ds(off15, 520)])


@functools.partial(
    pl.kernel,
    out_type=(jax.ShapeDtypeStruct((ROWS_PAD, H), F32),
              jax.ShapeDtypeStruct((ROWS_PAD, H), F32)),
    mesh=plsc.VectorSubcoreMesh(
        core_axis_name="c", subcore_axis_name="s", num_cores=2),
    scratch_types=dict(
        acc=pltpu.VMEM_SHARED((ROWS_PAD, H), F32),
        sbigA=pltpu.VMEM((BLK, C), jnp.int32),
        sbigB=pltpu.VMEM((BLK, C), jnp.int32),
        dbigA=pltpu.VMEM((BLK, C), jnp.int32),
        dbigB=pltpu.VMEM((BLK, C), jnp.int32),
        rows0=pltpu.VMEM((C, H), F32),
        rows1=pltpu.VMEM((C, H), F32),
        isemA=pltpu.SemaphoreType.DMA,
        isemB=pltpu.SemaphoreType.DMA,
        gsem0=pltpu.SemaphoreType.DMA,
        gsem1=pltpu.SemaphoreType.DMA,
    ),
)
def _hop_kernel(w_hbm, src2_hbm, dst2_hbm, zeros2_hbm, p0_out, p1_out,
                acc, sbigA, sbigB, dbigA, dbigB, rows0, rows1,
                isemA, isemB, gsem0, gsem1):
    cid = lax.axis_index("c")
    t = lax.axis_index("s")
    wid = cid * NTILES + t
    sbufs = (sbigA, sbigB)
    dbufs = (dbigA, dbigB)
    isems = (isemA, isemB)
    rbufs = (rows0, rows1)
    gsems = (gsem0, gsem1)

    def ifetch(b, sbig, dbig, isem):
        row0 = pl.multiple_of(wid * CHUNKS + b * BLK, 8)
        pltpu.async_copy(src2_hbm.at[pl.ds(row0, BLK)], sbig, isem)
        pltpu.async_copy(dst2_hbm.at[pl.ds(row0, BLK)], dbig, isem)

    ifetch(0, sbufs[0], dbufs[0], isems[0])
    pltpu.sync_copy(zeros2_hbm.at[pl.ds(0, ZROWS)],
                    acc.at[pl.ds(pl.multiple_of(t * ZROWS, 8), ZROWS)])
    plsc.subcore_barrier()


    def iwait(sbig, dbig, isem):
        pltpu.make_async_copy(src2_hbm.at[pl.ds(0, BLK)], sbig, isem).wait()
        pltpu.make_async_copy(dst2_hbm.at[pl.ds(0, BLK)], dbig, isem).wait()

    for b in range(NBLK):
        sbig, dbig, isem = sbufs[b % 2], dbufs[b % 2], isems[b % 2]
        iwait(sbig, dbig, isem)
        if b + 1 < NBLK:
            ifetch(b + 1, sbufs[(b + 1) % 2], dbufs[(b + 1) % 2],
                   isems[(b + 1) % 2])
        pltpu.async_copy(w_hbm.at[sbig.at[0]], rbufs[0], gsems[0])
        for k in range(BLK):
            rows, gsem = rbufs[k % 2], gsems[k % 2]
            if k + 1 < BLK:
                pltpu.async_copy(w_hbm.at[sbig.at[k + 1]],
                                 rbufs[(k + 1) % 2], gsems[(k + 1) % 2])
            pltpu.make_async_copy(
                w_hbm.at[pl.ds(0, C)], rows, gsem).wait()
            pltpu.sync_copy(rows, acc.at[dbig.at[k]], add=True)

    plsc.subcore_barrier()

    stripe = pl.ds(pl.multiple_of(t * ZROWS, 8), ZROWS)

    @pl.when(cid == 0)
    def _():
        pltpu.sync_copy(acc.at[stripe], p0_out.at[stripe])

    @pl.when(cid == 1)
    def _():
        pltpu.sync_copy(acc.at[stripe], p1_out.at[stripe])


def _embed_body(h_ref, we_ref, be_ref, d0_ref, d1_ref,
                hcur_ref, w_ref, rs_ref, rs2_ref):
    degc = jnp.maximum(d0_ref[...] + d1_ref[...], 1.0)
    rs = lax.rsqrt(degc)
    rs_ref[...] = rs
    rs2_ref[...] = 1.0 / degc
    hcur = jnp.dot(h_ref[...], we_ref[...],
                   preferred_element_type=F32) + be_ref[...]
    hcur_ref[...] = hcur
    w_ref[...] = hcur * rs


def _embed(h, We, be, d0_col, d1_col):
    return pl.pallas_call(
        _embed_body,
        out_shape=(jax.ShapeDtypeStruct((N, H), F32),
                   jax.ShapeDtypeStruct((N, H), F32),
                   jax.ShapeDtypeStruct((N, 1), F32),
                   jax.ShapeDtypeStruct((N, 1), F32)),
    )(h, We, be, d0_col, d1_col)


def _combine_body(p0_ref, p1_ref, rs2_ref, a_ref, w_ref):
    a = p0_ref[0:N, :] + p1_ref[0:N, :]
    a_ref[...] = a
    w_ref[...] = a * rs2_ref[...]


def _combine(p0, p1, rs2_col):
    return pl.pallas_call(
        _combine_body,
        out_shape=(jax.ShapeDtypeStruct((N, H), F32),
                   jax.ShapeDtypeStruct((N, H), F32)),
    )(p0, p1, rs2_col)


def _layer_body(hin_ref, a1_ref, a2_ref, p30_ref, p31_ref, rs_ref, sn_ref,
                A_ref, Wf_ref, bf_ref, g_ref, b_ref, hout_ref, wout_ref):
    hin = hin_ref[...]
    rs = rs_ref[...]
    third = 1.0 / 3.0

    def M(k):
        return (A_ref[0, k] * Wf_ref[0] + A_ref[1, k] * Wf_ref[1]
                + A_ref[2, k] * Wf_ref[2]) * third

    hc = jnp.dot(hin, M(0), preferred_element_type=F32)
    hc = hc + jnp.dot(a1_ref[...] * rs, M(1), preferred_element_type=F32)
    hc = hc + jnp.dot(a2_ref[...] * rs, M(2), preferred_element_type=F32)
    hc = hc + jnp.dot((p30_ref[0:N, :] + p31_ref[0:N, :]) * rs, M(3),
                      preferred_element_type=F32)
    bbar = (bf_ref[0] + bf_ref[1] + bf_ref[2]) * third
    hc = (hc + bbar) * sn_ref[...]
    mu = jnp.mean(hc, axis=0, keepdims=True)
    var = jnp.mean((hc - mu) ** 2, axis=0, keepdims=True)
    hc = (hc - mu) * lax.rsqrt(var + 1e-5) * g_ref[...] + b_ref[...]
    hc = jnp.maximum(hc, 0.0)
    hout = hc + hin
    hout_ref[...] = hout
    wout_ref[...] = hout * rs


def _layer(hin, a1, a2, p30, p31, rs_col, snorm_n, A_l, Wf_l, bf_l, g_l,
           b_l):
    return pl.pallas_call(
        _layer_body,
        out_shape=(jax.ShapeDtypeStruct((N, H), F32),
                   jax.ShapeDtypeStruct((N, H), F32)),
        in_specs=[pl.BlockSpec(memory_space=pltpu.MemorySpace.VMEM)] * 7
        + [pl.BlockSpec(memory_space=pltpu.MemorySpace.SMEM)]
        + [pl.BlockSpec(memory_space=pltpu.MemorySpace.VMEM)] * 4,
        compiler_params=pltpu.CompilerParams(
            vmem_limit_bytes=63 * 1024 * 1024),
    )(hin, a1, a2, p30, p31, rs_col, snorm_n, A_l, Wf_l, bf_l, g_l, b_l)


def _readout_body(hc_ref, w1_ref, b1_ref, w2_ref, b2_ref, w3_ref, b3_ref,
                  out_ref):
    hg = jnp.mean(hc_ref[...], axis=0, keepdims=True)
    x = jnp.maximum(jnp.dot(hg, w1_ref[...],
                            preferred_element_type=F32) + b1_ref[...], 0.0)
    x = jnp.maximum(jnp.dot(x, w2_ref[...],
                            preferred_element_type=F32) + b2_ref[...], 0.0)
    out_ref[...] = jnp.dot(x, w3_ref[...],
                           preferred_element_type=F32) + b3_ref[...]


def _readout(hc, W1, b1, W2, b2, W3, b3):
    return pl.pallas_call(
        _readout_body,
        out_shape=jax.ShapeDtypeStruct((1, 10), F32),
    )(hc, W1, b1, W2, b2, W3, b3)


def kernel(h, edge_index, e, snorm_n, snorm_e, W_embed, b_embed, A_coef,
           Wf, bf, gamma, beta, W1, b1, W2, b2, W3, b3):
    src = edge_index[0]
    dst = edge_index[1]
    i32 = jnp.int32
    src2 = jnp.concatenate(
        [src, jnp.zeros((E_PAD - E,), i32)]).reshape(CROWS, C)
    dst2 = jnp.concatenate(
        [dst, jnp.full((E_PAD - E,), N, i32)]).reshape(CROWS, C)
    zeros1 = jnp.zeros((DEG_PAD,), F32)
    zeros2 = jnp.zeros((ZROWS, H), F32)

    d0, d1 = _deg_kernel(dst2, zeros1)
    hcur, w, rs_col, rs2_col = _embed(
        h, W_embed, b_embed.reshape(1, H), d0[:, None], d1[:, None])
    for l in range(4):
        hin = hcur
        a_list = []
        for _ in range(2):
            p0, p1 = _hop_kernel(w, src2, dst2, zeros2)
            a, w = _combine(p0, p1, rs2_col)
            a_list.append(a)
        p30, p31 = _hop_kernel(w, src2, dst2, zeros2)
        hcur, w = _layer(hin, a_list[0], a_list[1], p30, p31, rs_col,
                         snorm_n, A_coef[l], Wf[l], bf[l].reshape(3, 1, H),
                         gamma[l].reshape(1, H), beta[l].reshape(1, H))
    out = _readout(hcur, W1, b1.reshape(1, -1), W2, b2.reshape(1, -1),
                   W3, b3.reshape(1, -1))
    return out


# speedup vs baseline: 4.0048x; 1.0864x over previous
import functools

import jax
import jax.numpy as jnp
from jax import lax
from jax.experimental import pallas as pl
from jax.experimental.pallas import tpu as pltpu
from jax.experimental.pallas import tpu_sc as plsc

N = 10000
E = 320000
H = 128
NTILES = 16
NW = 32
C = 80
CHUNKS = 128
E_PAD = NW * C * CHUNKS
CROWS = NW * CHUNKS
BLK = 16
NBLK = CHUNKS // BLK
NRB = 4
ROWS_PAD = 10112
ZROWS = 632
DEG_PAD = 10240
F32 = jnp.float32


@functools.partial(
    pl.kernel,
    out_type=(jax.ShapeDtypeStruct((N,), F32),
              jax.ShapeDtypeStruct((N,), F32)),
    mesh=plsc.VectorSubcoreMesh(
        core_axis_name="c", subcore_axis_name="s", num_cores=2),
    scratch_types=dict(
        acc=pltpu.VMEM_SHARED((DEG_PAD,), F32),
        dbigA=pltpu.VMEM((BLK, C), jnp.int32),
        dbigB=pltpu.VMEM((BLK, C), jnp.int32),
        ones=pltpu.VMEM((C,), F32),
        dbuf=pltpu.VMEM((ZROWS,), F32),
        isemA=pltpu.SemaphoreType.DMA,
        isemB=pltpu.SemaphoreType.DMA,
        ssem=pltpu.SemaphoreType.DMA,
    ),
)
def _deg_kernel(dst2_hbm, zeros1_hbm, deg0_out, deg1_out,
                acc, dbigA, dbigB, ones, dbuf, isemA, isemB, ssem):
    cid = lax.axis_index("c")
    t = lax.axis_index("s")
    wid = cid * NTILES + t
    pltpu.sync_copy(zeros1_hbm.at[pl.ds(t * 640, 640)],
                    acc.at[pl.ds(t * 640, 640)])
    for j in range(C // 16):
        ones[pl.ds(j * 16, 16)] = jnp.ones((16,), F32)
    plsc.subcore_barrier()

    bufs = (dbigA, dbigB)
    sems = (isemA, isemB)

    def ifetch(b, dbig, isem):
        row0 = pl.multiple_of(wid * CHUNKS + b * BLK, 8)
        pltpu.async_copy(dst2_hbm.at[pl.ds(row0, BLK)], dbig, isem)

    ifetch(0, bufs[0], sems[0])
    for b in range(NBLK):
        dbig, isem = bufs[b % 2], sems[b % 2]
        pltpu.make_async_copy(
            dst2_hbm.at[pl.ds(0, BLK)], dbig, isem).wait()
        if b + 1 < NBLK:
            ifetch(b + 1, bufs[(b + 1) % 2], sems[(b + 1) % 2])
        for k in range(BLK):
            pltpu.async_copy(ones, acc.at[dbig.at[k]], ssem, add=True)
        for k in range(BLK):
            pltpu.make_async_copy(ones, acc.at[dbig.at[0]], ssem).wait()

    plsc.subcore_barrier()
    off = pl.multiple_of(t * ZROWS, 8)

    @pl.when(t < NTILES - 1)
    def _():
        pltpu.sync_copy(acc.at[pl.ds(off, ZROWS)], dbuf)

        @pl.when(cid == 0)
        def _():
            pltpu.sync_copy(dbuf, deg0_out.at[pl.ds(off, ZROWS)])

        @pl.when(cid == 1)
        def _():
            pltpu.sync_copy(dbuf, deg1_out.at[pl.ds(off, ZROWS)])

    @pl.when(t == NTILES - 1)
    def _():
        off15 = pl.multiple_of((NTILES - 1) * ZROWS, 8)
        pltpu.sync_copy(acc.at[pl.ds(off15, 520)], dbuf.at[pl.ds(0, 520)])

        @pl.when(cid == 0)
        def _():
            pltpu.sync_copy(dbuf.at[pl.ds(0, 520)],
                            deg0_out.at[pl.ds(off15, 520)])

        @pl.when(cid == 1)
        def _():
            pltpu.sync_copy(dbuf.at[pl.ds(0, 520)],
                            deg1_out.at[pl.ds(off15, 520)])


@functools.partial(
    pl.kernel,
    out_type=(jax.ShapeDtypeStruct((ROWS_PAD, H), F32),
              jax.ShapeDtypeStruct((ROWS_PAD, H), F32)),
    mesh=plsc.VectorSubcoreMesh(
        core_axis_name="c", subcore_axis_name="s", num_cores=2),
    scratch_types=dict(
        acc=pltpu.VMEM_SHARED((ROWS_PAD, H), F32),
        sidx0=pltpu.VMEM((C,), jnp.int32),
        sidx1=pltpu.VMEM((C,), jnp.int32),
        sidx2=pltpu.VMEM((C,), jnp.int32),
        sidx3=pltpu.VMEM((C,), jnp.int32),
        didx0=pltpu.VMEM((C,), jnp.int32),
        didx1=pltpu.VMEM((C,), jnp.int32),
        didx2=pltpu.VMEM((C,), jnp.int32),
        didx3=pltpu.VMEM((C,), jnp.int32),
        rows0=pltpu.VMEM((C, H), F32),
        rows1=pltpu.VMEM((C, H), F32),
        rows2=pltpu.VMEM((C, H), F32),
        rows3=pltpu.VMEM((C, H), F32),
        gsem0=pltpu.SemaphoreType.DMA,
        gsem1=pltpu.SemaphoreType.DMA,
        gsem2=pltpu.SemaphoreType.DMA,
        gsem3=pltpu.SemaphoreType.DMA,
        ssem0=pltpu.SemaphoreType.DMA,
        ssem1=pltpu.SemaphoreType.DMA,
        ssem2=pltpu.SemaphoreType.DMA,
        ssem3=pltpu.SemaphoreType.DMA,
    ),
)
def _hop_kernel(w_hbm, srcF_hbm, dstF_hbm, zeros2_hbm, p0_out, p1_out,
                acc, sidx0, sidx1, sidx2, sidx3, didx0, didx1, didx2,
                didx3, rows0, rows1, rows2, rows3, gsem0, gsem1, gsem2,
                gsem3, ssem0, ssem1, ssem2, ssem3):
    cid = lax.axis_index("c")
    t = lax.axis_index("s")
    wid = cid * NTILES + t
    sidx = (sidx0, sidx1, sidx2, sidx3)
    didx = (didx0, didx1, didx2, didx3)
    rows = (rows0, rows1, rows2, rows3)
    gsem = (gsem0, gsem1, gsem2, gsem3)
    ssem = (ssem0, ssem1, ssem2, ssem3)

    pltpu.sync_copy(zeros2_hbm.at[pl.ds(0, ZROWS)],
                    acc.at[pl.ds(pl.multiple_of(t * ZROWS, 8), ZROWS)])
    plsc.subcore_barrier()

    def fetch(c, j):
        base = pl.multiple_of(c * C, C)
        pltpu.sync_copy(srcF_hbm.at[pl.ds(base, C)], sidx[j])
        pltpu.sync_copy(dstF_hbm.at[pl.ds(base, C)], didx[j])
        pltpu.async_copy(w_hbm.at[sidx[j]], rows[j], gsem[j])

    def gwait(j):
        pltpu.make_async_copy(w_hbm.at[pl.ds(0, C)], rows[j],
                              gsem[j]).wait()

    def swait(j):
        pltpu.make_async_copy(w_hbm.at[pl.ds(0, C)], rows[j],
                              ssem[j]).wait()

    def scatter(j):
        pltpu.async_copy(rows[j], acc.at[didx[j]], ssem[j], add=True)

    c0 = wid * CHUNKS
    fetch(c0, 0)
    fetch(c0 + 1, 1)

    def body(i2, carry):
        c = c0 + i2 * 4
        last = i2 == (CHUNKS // 4) - 1
        first = i2 == 0
        gwait(0)
        scatter(0)

        @pl.when(jnp.logical_not(first))
        def _():
            swait(2)

        fetch(c + 2, 2)
        gwait(1)
        scatter(1)

        @pl.when(jnp.logical_not(first))
        def _():
            swait(3)

        fetch(c + 3, 3)
        gwait(2)
        scatter(2)

        @pl.when(jnp.logical_not(last))
        def _():
            swait(0)
            fetch(c + 4, 0)

        gwait(3)
        scatter(3)

        @pl.when(jnp.logical_not(last))
        def _():
            swait(1)
            fetch(c + 5, 1)

        return carry

    lax.fori_loop(0, CHUNKS // 4, body, 0)
    for j in range(NRB):
        swait(j)

    plsc.subcore_barrier()

    stripe = pl.ds(pl.multiple_of(t * ZROWS, 8), ZROWS)

    @pl.when(cid == 0)
    def _():
        pltpu.sync_copy(acc.at[stripe], p0_out.at[stripe])

    @pl.when(cid == 1)
    def _():
        pltpu.sync_copy(acc.at[stripe], p1_out.at[stripe])


def _embed_body(h_ref, we_ref, be_ref, d0_ref, d1_ref,
                hcur_ref, w_ref, rs_ref, rs2_ref):
    degc = jnp.maximum(d0_ref[...] + d1_ref[...], 1.0)
    rs = lax.rsqrt(degc)
    rs_ref[...] = rs
    rs2_ref[...] = 1.0 / degc
    hcur = jnp.dot(h_ref[...], we_ref[...],
                   preferred_element_type=F32) + be_ref[...]
    hcur_ref[...] = hcur
    w_ref[...] = hcur * rs


def _embed(h, We, be, d0_col, d1_col):
    return pl.pallas_call(
        _embed_body,
        out_shape=(jax.ShapeDtypeStruct((N, H), F32),
                   jax.ShapeDtypeStruct((N, H), F32),
                   jax.ShapeDtypeStruct((N, 1), F32),
                   jax.ShapeDtypeStruct((N, 1), F32)),
    )(h, We, be, d0_col, d1_col)


def _combine_body(p0_ref, p1_ref, rs2_ref, a_ref, w_ref):
    a = p0_ref[0:N, :] + p1_ref[0:N, :]
    a_ref[...] = a
    w_ref[...] = a * rs2_ref[...]


def _combine(p0, p1, rs2_col):
    return pl.pallas_call(
        _combine_body,
        out_shape=(jax.ShapeDtypeStruct((N, H), F32),
                   jax.ShapeDtypeStruct((N, H), F32)),
    )(p0, p1, rs2_col)


def _layer_body(hin_ref, a1_ref, a2_ref, p30_ref, p31_ref, rs_ref, sn_ref,
                A_ref, Wf_ref, bf_ref, g_ref, b_ref, hout_ref, wout_ref):
    hin = hin_ref[...]
    rs = rs_ref[...]
    third = 1.0 / 3.0

    def M(k):
        return (A_ref[0, k] * Wf_ref[0] + A_ref[1, k] * Wf_ref[1]
                + A_ref[2, k] * Wf_ref[2]) * third

    hc = jnp.dot(hin, M(0), preferred_element_type=F32)
    hc = hc + jnp.dot(a1_ref[...] * rs, M(1), preferred_element_type=F32)
    hc = hc + jnp.dot(a2_ref[...] * rs, M(2), preferred_element_type=F32)
    hc = hc + jnp.dot((p30_ref[0:N, :] + p31_ref[0:N, :]) * rs, M(3),
                      preferred_element_type=F32)
    bbar = (bf_ref[0] + bf_ref[1] + bf_ref[2]) * third
    hc = (hc + bbar) * sn_ref[...]
    mu = jnp.mean(hc, axis=0, keepdims=True)
    var = jnp.mean((hc - mu) ** 2, axis=0, keepdims=True)
    hc = (hc - mu) * lax.rsqrt(var + 1e-5) * g_ref[...] + b_ref[...]
    hc = jnp.maximum(hc, 0.0)
    hout = hc + hin
    hout_ref[...] = hout
    wout_ref[...] = hout * rs


def _layer(hin, a1, a2, p30, p31, rs_col, snorm_n, A_l, Wf_l, bf_l, g_l,
           b_l):
    return pl.pallas_call(
        _layer_body,
        out_shape=(jax.ShapeDtypeStruct((N, H), F32),
                   jax.ShapeDtypeStruct((N, H), F32)),
        in_specs=[pl.BlockSpec(memory_space=pltpu.MemorySpace.VMEM)] * 7
        + [pl.BlockSpec(memory_space=pltpu.MemorySpace.SMEM)]
        + [pl.BlockSpec(memory_space=pltpu.MemorySpace.VMEM)] * 4,
        compiler_params=pltpu.CompilerParams(
            vmem_limit_bytes=63 * 1024 * 1024),
    )(hin, a1, a2, p30, p31, rs_col, snorm_n, A_l, Wf_l, bf_l, g_l, b_l)


def _readout_body(hc_ref, w1_ref, b1_ref, w2_ref, b2_ref, w3_ref, b3_ref,
                  out_ref):
    hg = jnp.mean(hc_ref[...], axis=0, keepdims=True)
    x = jnp.maximum(jnp.dot(hg, w1_ref[...],
                            preferred_element_type=F32) + b1_ref[...], 0.0)
    x = jnp.maximum(jnp.dot(x, w2_ref[...],
                            preferred_element_type=F32) + b2_ref[...], 0.0)
    out_ref[...] = jnp.dot(x, w3_ref[...],
                           preferred_element_type=F32) + b3_ref[...]


def _readout(hc, W1, b1, W2, b2, W3, b3):
    return pl.pallas_call(
        _readout_body,
        out_shape=jax.ShapeDtypeStruct((1, 10), F32),
    )(hc, W1, b1, W2, b2, W3, b3)


def kernel(h, edge_index, e, snorm_n, snorm_e, W_embed, b_embed, A_coef,
           Wf, bf, gamma, beta, W1, b1, W2, b2, W3, b3):
    src = edge_index[0]
    dst = edge_index[1]
    i32 = jnp.int32
    srcF = jnp.concatenate([src, jnp.zeros((E_PAD - E,), i32)])
    dstF = jnp.concatenate([dst, jnp.full((E_PAD - E,), N, i32)])
    dst2 = dstF.reshape(CROWS, C)
    zeros1 = jnp.zeros((DEG_PAD,), F32)
    zeros2 = jnp.zeros((ZROWS, H), F32)

    d0, d1 = _deg_kernel(dst2, zeros1)
    hcur, w, rs_col, rs2_col = _embed(
        h, W_embed, b_embed.reshape(1, H), d0[:, None], d1[:, None])
    for l in range(4):
        hin = hcur
        a_list = []
        for _ in range(2):
            p0, p1 = _hop_kernel(w, srcF, dstF, zeros2)
            a, w = _combine(p0, p1, rs2_col)
            a_list.append(a)
        p30, p31 = _hop_kernel(w, srcF, dstF, zeros2)
        hcur, w = _layer(hin, a_list[0], a_list[1], p30, p31, rs_col,
                         snorm_n, A_coef[l], Wf[l], bf[l].reshape(3, 1, H),
                         gamma[l].reshape(1, H), beta[l].reshape(1, H))
    out = _readout(hcur, W1, b1.reshape(1, -1), W2, b2.reshape(1, -1),
                   W3, b3.reshape(1, -1))
    return out
